# Initial kernel scaffold; baseline (speedup 1.0000x reference)
import jax
import jax.numpy as jnp
from jax.experimental import pallas as pl


def kernel(x, edge_index, W, b):
    raise NotImplementedError("write your pallas kernel here")


# speedup vs baseline: 1.6555x; 1.6555x over previous
import functools

import jax
import jax.numpy as jnp
from jax import lax
from jax.experimental import pallas as pl
from jax.experimental.pallas import tpu as pltpu
from jax.experimental.pallas import tpu_sc as plsc

N_NODES = 10000
C = 128
K = 32
N_WORKERS = 32
NODES_PER_W = 320
N_PAD = N_WORKERS * NODES_PER_W
CHUNK_NODES = 4
CHUNKS = NODES_PER_W // CHUNK_NODES
LANES = 16
COLS = C // LANES

MM_BLOCK = 1280


def _mm_body(x_ref, w_ref, bias_ref, a_ref, b_ref):
    xb = x_ref[...]
    w1 = w_ref[:, :C]
    w2 = w_ref[:, C:]
    dn = (((1,), (1,)), ((), ()))
    a_ref[...] = lax.dot_general(xb, w1 - w2, dn,
                                 preferred_element_type=jnp.float32) + bias_ref[...]
    b_ref[...] = lax.dot_general(xb, w2, dn,
                                 preferred_element_type=jnp.float32)


@jax.jit
def _mm_call(x_pad, W, bias):
    grid = (N_PAD // MM_BLOCK,)
    return pl.pallas_call(
        _mm_body,
        grid=grid,
        in_specs=[
            pl.BlockSpec((MM_BLOCK, C), lambda i: (i, 0)),
            pl.BlockSpec((C, 2 * C), lambda i: (0, 0)),
            pl.BlockSpec((1, C), lambda i: (0, 0)),
        ],
        out_specs=[
            pl.BlockSpec((MM_BLOCK, C), lambda i: (i, 0)),
            pl.BlockSpec((MM_BLOCK, C), lambda i: (i, 0)),
        ],
        out_shape=[
            jax.ShapeDtypeStruct((N_PAD, C), jnp.float32),
            jax.ShapeDtypeStruct((N_PAD, C), jnp.float32),
        ],
    )(x_pad, W, bias)


def _sc_body(a_hbm, b_hbm, idx_hbm, out_hbm, idx_v, a_v, g_v, o_v, sem):
    wid = lax.axis_index("s") * 2 + lax.axis_index("c")
    base = wid * NODES_PER_W
    pltpu.sync_copy(idx_hbm.at[wid], idx_v)
    pltpu.sync_copy(a_hbm.at[pl.ds(base, NODES_PER_W)], a_v)

    def chunk_body(c, carry):
        pltpu.async_copy(b_hbm.at[idx_v.at[c]], g_v, sem).wait()
        for n in range(CHUNK_NODES):
            row = c * CHUNK_NODES + n
            for col in range(COLS):
                sl = pl.ds(col * LANES, LANES)
                acc = g_v[n * K, sl]
                for k in range(1, K):
                    acc = jnp.maximum(acc, g_v[n * K + k, sl])
                z = a_v[row, sl] + acc
                o_v[row, sl] = jnp.where(z > 0, z, jnp.exp(z) - 1.0)
        return carry

    lax.fori_loop(0, CHUNKS, chunk_body, 0)
    pltpu.sync_copy(o_v, out_hbm.at[pl.ds(base, NODES_PER_W)])


@jax.jit
def _sc_call(A, B, idx):
    mesh = plsc.VectorSubcoreMesh(core_axis_name="c", subcore_axis_name="s")
    f = functools.partial(
        pl.kernel,
        out_type=jax.ShapeDtypeStruct((N_PAD, C), jnp.float32),
        mesh=mesh,
        scratch_types=[
            pltpu.VMEM((CHUNKS, CHUNK_NODES * K), jnp.int32),
            pltpu.VMEM((NODES_PER_W, C), jnp.float32),
            pltpu.VMEM((CHUNK_NODES * K, C), jnp.float32),
            pltpu.VMEM((NODES_PER_W, C), jnp.float32),
            pltpu.SemaphoreType.DMA,
        ],
    )(_sc_body)
    return f(A, B, idx)


def kernel(x, edge_index, W, b):
    x = x.astype(jnp.float32)
    ei = edge_index.astype(jnp.int32)
    x_pad = jnp.concatenate([x, jnp.zeros((N_PAD - N_NODES, C), jnp.float32)], axis=0)
    A, B = _mm_call(x_pad, W, b.reshape(1, C))
    ei_pad = jnp.concatenate(
        [ei, jnp.zeros((N_PAD - N_NODES, K), jnp.int32)], axis=0)
    idx = ei_pad.reshape(N_WORKERS, CHUNKS, CHUNK_NODES * K)
    out_pad = _sc_call(A, B, idx)
    return out_pad[:N_NODES]


# speedup vs baseline: 1.9993x; 1.2076x over previous
import functools

import jax
import jax.numpy as jnp
from jax import lax
from jax.experimental import pallas as pl
from jax.experimental.pallas import tpu as pltpu
from jax.experimental.pallas import tpu_sc as plsc

N_NODES = 10000
C = 128
K = 32
N_WORKERS = 32
NODES_PER_W = 320
N_PAD = N_WORKERS * NODES_PER_W
CHUNK_NODES = 4
CHUNKS = NODES_PER_W // CHUNK_NODES
LANES = 16
COLS = C // LANES

MM_BLOCK = 1280


def _mm_body(x_ref, w_ref, bias_ref, a_ref, b_ref):
    xb = x_ref[...]
    w1 = w_ref[:, :C]
    w2 = w_ref[:, C:]
    dn = (((1,), (1,)), ((), ()))
    a_ref[...] = lax.dot_general(xb, w1 - w2, dn,
                                 preferred_element_type=jnp.float32) + bias_ref[...]
    b_ref[...] = lax.dot_general(xb, w2, dn,
                                 preferred_element_type=jnp.float32)


@jax.jit
def _mm_call(x_pad, W, bias):
    grid = (N_PAD // MM_BLOCK,)
    return pl.pallas_call(
        _mm_body,
        grid=grid,
        in_specs=[
            pl.BlockSpec((MM_BLOCK, C), lambda i: (i, 0)),
            pl.BlockSpec((C, 2 * C), lambda i: (0, 0)),
            pl.BlockSpec((1, C), lambda i: (0, 0)),
        ],
        out_specs=[
            pl.BlockSpec((MM_BLOCK, C), lambda i: (i, 0)),
            pl.BlockSpec((MM_BLOCK, C), lambda i: (i, 0)),
        ],
        out_shape=[
            jax.ShapeDtypeStruct((N_PAD, C), jnp.float32),
            jax.ShapeDtypeStruct((N_PAD, C), jnp.float32),
        ],
    )(x_pad, W, bias)


def _sc_body(a_hbm, b_hbm, idx_hbm, out_hbm, idx_v, a_v, g0, g1, o_v, sem0, sem1):
    wid = lax.axis_index("s") * 2 + lax.axis_index("c")
    base = wid * NODES_PER_W
    pltpu.sync_copy(idx_hbm.at[wid], idx_v)
    pltpu.sync_copy(a_hbm.at[pl.ds(base, NODES_PER_W)], a_v)

    bufs = (g0, g1)
    sems = (sem0, sem1)

    def start(c, buf, sm):
        pltpu.make_async_copy(b_hbm.at[idx_v.at[c]], buf, sm).start()

    def wait(buf, sm):
        pltpu.make_async_copy(b_hbm.at[idx_v.at[0]], buf, sm).wait()

    def compute(c, buf):
        for n in range(CHUNK_NODES):
            row = c * CHUNK_NODES + n
            for col in range(COLS):
                sl = pl.ds(col * LANES, LANES)
                vals = [buf[n * K + k, sl] for k in range(K)]
                while len(vals) > 1:
                    vals = [jnp.maximum(vals[i], vals[i + 1])
                            for i in range(0, len(vals), 2)]
                z = a_v[row, sl] + vals[0]
                o_v[row, sl] = jnp.where(z > 0, z, jnp.exp(z) - 1.0)

    start(0, g0, sem0)

    def pair_body(i, carry):
        g = i * 2
        for par in range(2):
            c = g + par
            @pl.when(c + 1 < CHUNKS)
            def _():
                start(c + 1, bufs[1 - par], sems[1 - par])
            wait(bufs[par], sems[par])
            compute(c, bufs[par])
        return carry

    lax.fori_loop(0, CHUNKS // 2, pair_body, 0)
    pltpu.sync_copy(o_v, out_hbm.at[pl.ds(base, NODES_PER_W)])


@jax.jit
def _sc_call(A, B, idx):
    mesh = plsc.VectorSubcoreMesh(core_axis_name="c", subcore_axis_name="s")
    f = functools.partial(
        pl.kernel,
        out_type=jax.ShapeDtypeStruct((N_PAD, C), jnp.float32),
        mesh=mesh,
        scratch_types=[
            pltpu.VMEM((CHUNKS, CHUNK_NODES * K), jnp.int32),
            pltpu.VMEM((NODES_PER_W, C), jnp.float32),
            pltpu.VMEM((CHUNK_NODES * K, C), jnp.float32),
            pltpu.VMEM((CHUNK_NODES * K, C), jnp.float32),
            pltpu.VMEM((NODES_PER_W, C), jnp.float32),
            pltpu.SemaphoreType.DMA,
            pltpu.SemaphoreType.DMA,
        ],
    )(_sc_body)
    return f(A, B, idx)


def kernel(x, edge_index, W, b):
    x = x.astype(jnp.float32)
    ei = edge_index.astype(jnp.int32)
    x_pad = jnp.concatenate([x, jnp.zeros((N_PAD - N_NODES, C), jnp.float32)], axis=0)
    A, B = _mm_call(x_pad, W, b.reshape(1, C))
    ei_pad = jnp.concatenate(
        [ei, jnp.zeros((N_PAD - N_NODES, K), jnp.int32)], axis=0)
    idx = ei_pad.reshape(N_WORKERS, CHUNKS, CHUNK_NODES * K)
    out_pad = _sc_call(A, B, idx)
    return out_pad[:N_NODES]
